# baseline (speedup 1.0000x reference)
import jax
import jax.numpy as jnp
from jax import lax
from jax.experimental import pallas as pl
from jax.experimental.pallas import tpu as pltpu

_G = 64
_CH = 2048


def _fused_body(x_ref, b_ref, W1_ref, b1_ref, W2_ref, b2_ref,
                out_ref, E_ref, S_ref, st_ref, en_ref):
    i = pl.program_id(0)
    nb = pl.num_programs(0)
    BLK = x_ref.shape[0]

    @pl.when(i == 0)
    def _init():
        out_ref[...] = jnp.zeros_like(out_ref)
        E_ref[...] = jnp.zeros_like(E_ref)
        S_ref[...] = jnp.zeros_like(S_ref)
        giota = lax.broadcasted_iota(jnp.int32, (_G, 1), 0)
        nch = b_ref.shape[0]
        cnt = jnp.zeros((_G, 1), jnp.float32)
        for c in range(nch):
            bc = b_ref[pl.ds(c, 1), :]
            eq = (giota == bc).astype(jnp.float32)
            cnt = cnt + jnp.sum(eq, axis=1, keepdims=True)
        gk = lax.broadcasted_iota(jnp.int32, (_G, _G), 0)
        gg = lax.broadcasted_iota(jnp.int32, (_G, _G), 1)
        tri_lt = (gk < gg).astype(jnp.float32)
        tri_le = (gk <= gg).astype(jnp.float32)
        st_ref[...] = jnp.sum(tri_lt * cnt, axis=0, keepdims=True)
        en_ref[...] = jnp.sum(tri_le * cnt, axis=0, keepdims=True)

    xb = x_ref[...]
    h = jnp.tanh(jnp.dot(xb, W1_ref[...],
                         preferred_element_type=jnp.float32) + b1_ref[...])
    s = jnp.dot(h, W2_ref[...],
                preferred_element_type=jnp.float32) + b2_ref[...]
    e = jnp.exp(s)
    r = (jnp.float32(1.0) * i * BLK
         + lax.broadcasted_iota(jnp.int32, (BLK, _G), 0).astype(jnp.float32))
    onehot = ((r >= st_ref[...]) & (r < en_ref[...])).astype(jnp.float32)
    oe = onehot * e
    out_ref[...] += lax.dot_general(
        oe, xb, (((0,), (0,)), ((), ())),
        preferred_element_type=jnp.float32)
    S_ref[...] += lax.dot_general(
        onehot, s, (((0,), (0,)), ((), ())),
        preferred_element_type=jnp.float32)
    E_ref[...] += lax.dot_general(
        onehot, e, (((0,), (0,)), ((), ())),
        preferred_element_type=jnp.float32)

    @pl.when(i == nb - 1)
    def _fin():
        em = jnp.exp(-S_ref[...])
        out_ref[...] = (em * out_ref[...]) / (em * E_ref[...] + 1e-8)


def kernel(x, batch, W1, b1, W2, b2):
    N, D = x.shape
    H = W1.shape[1]
    BLK = 2000
    assert N % BLK == 0
    nb = N // BLK
    npad = -N % _CH
    b2d = jnp.pad(batch.astype(jnp.int32), (0, npad),
                  constant_values=jnp.int32(2 ** 30)).reshape(-1, _CH)
    nch = b2d.shape[0]
    return pl.pallas_call(
        _fused_body,
        grid=(nb,),
        in_specs=[
            pl.BlockSpec((BLK, D), lambda i: (i, 0)),
            pl.BlockSpec((nch, _CH), lambda i: (0, 0)),
            pl.BlockSpec((D, H), lambda i: (0, 0)),
            pl.BlockSpec((1, H), lambda i: (0, 0)),
            pl.BlockSpec((H, 1), lambda i: (0, 0)),
            pl.BlockSpec((1, 1), lambda i: (0, 0)),
        ],
        out_specs=pl.BlockSpec((_G, D), lambda i: (0, 0)),
        out_shape=jax.ShapeDtypeStruct((_G, D), jnp.float32),
        scratch_shapes=[
            pltpu.VMEM((_G, 1), jnp.float32),
            pltpu.VMEM((_G, 1), jnp.float32),
            pltpu.VMEM((1, _G), jnp.float32),
            pltpu.VMEM((1, _G), jnp.float32),
        ],
    )(x, b2d, W1, b1.reshape(1, H), W2, b2.reshape(1, 1))


# speedup vs baseline: 1.0570x; 1.0570x over previous
import jax
import jax.numpy as jnp
from jax import lax
from jax.experimental import pallas as pl
from jax.experimental.pallas import tpu as pltpu

_G = 64
_CH = 2048


def _fused_body(x_ref, b_ref, W1_ref, b1_ref, W2_ref, b2_ref,
                out_ref, E_ref, S_ref, st_ref, en_ref):
    i = pl.program_id(0)
    nb = pl.num_programs(0)
    BLK = x_ref.shape[0]

    @pl.when(i == 0)
    def _init():
        out_ref[...] = jnp.zeros_like(out_ref)
        E_ref[...] = jnp.zeros_like(E_ref)
        S_ref[...] = jnp.zeros_like(S_ref)
        giota = lax.broadcasted_iota(jnp.int32, (_G, 1), 0)
        nch = b_ref.shape[0]
        cnt = jnp.zeros((_G, 1), jnp.float32)
        for c in range(nch):
            bc = b_ref[pl.ds(c, 1), :]
            eq = (giota == bc).astype(jnp.float32)
            cnt = cnt + jnp.sum(eq, axis=1, keepdims=True)
        gk = lax.broadcasted_iota(jnp.int32, (_G, _G), 0)
        gg = lax.broadcasted_iota(jnp.int32, (_G, _G), 1)
        tri_lt = (gk < gg).astype(jnp.float32)
        tri_le = (gk <= gg).astype(jnp.float32)
        st_ref[...] = jnp.sum(tri_lt * cnt, axis=0, keepdims=True)
        en_ref[...] = jnp.sum(tri_le * cnt, axis=0, keepdims=True)

    xb = x_ref[...]
    xb16 = xb.astype(jnp.bfloat16)
    h = jnp.tanh(jnp.dot(xb16, W1_ref[...].astype(jnp.bfloat16),
                         preferred_element_type=jnp.float32) + b1_ref[...])
    s = jnp.dot(h.astype(jnp.bfloat16), W2_ref[...].astype(jnp.bfloat16),
                preferred_element_type=jnp.float32) + b2_ref[...]
    e = jnp.exp(s)
    r = (jnp.float32(1.0) * i * BLK
         + lax.broadcasted_iota(jnp.int32, (BLK, _G), 0).astype(jnp.float32))
    onehot = ((r >= st_ref[...]) & (r < en_ref[...])).astype(jnp.float32)
    oe = onehot * e
    out_ref[...] += lax.dot_general(
        oe, xb, (((0,), (0,)), ((), ())),
        preferred_element_type=jnp.float32)
    S_ref[...] += lax.dot_general(
        onehot, s, (((0,), (0,)), ((), ())),
        preferred_element_type=jnp.float32)
    E_ref[...] += lax.dot_general(
        onehot, e, (((0,), (0,)), ((), ())),
        preferred_element_type=jnp.float32)

    @pl.when(i == nb - 1)
    def _fin():
        em = jnp.exp(-S_ref[...])
        out_ref[...] = (em * out_ref[...]) / (em * E_ref[...] + 1e-8)


def kernel(x, batch, W1, b1, W2, b2):
    N, D = x.shape
    H = W1.shape[1]
    BLK = 5000
    assert N % BLK == 0
    nb = N // BLK
    npad = -N % _CH
    b2d = jnp.pad(batch.astype(jnp.int32), (0, npad),
                  constant_values=jnp.int32(2 ** 30)).reshape(-1, _CH)
    nch = b2d.shape[0]
    return pl.pallas_call(
        _fused_body,
        grid=(nb,),
        in_specs=[
            pl.BlockSpec((BLK, D), lambda i: (i, 0)),
            pl.BlockSpec((nch, _CH), lambda i: (0, 0)),
            pl.BlockSpec((D, H), lambda i: (0, 0)),
            pl.BlockSpec((1, H), lambda i: (0, 0)),
            pl.BlockSpec((H, 1), lambda i: (0, 0)),
            pl.BlockSpec((1, 1), lambda i: (0, 0)),
        ],
        out_specs=pl.BlockSpec((_G, D), lambda i: (0, 0)),
        out_shape=jax.ShapeDtypeStruct((_G, D), jnp.float32),
        scratch_shapes=[
            pltpu.VMEM((_G, 1), jnp.float32),
            pltpu.VMEM((_G, 1), jnp.float32),
            pltpu.VMEM((1, _G), jnp.float32),
            pltpu.VMEM((1, _G), jnp.float32),
        ],
    )(x, b2d, W1, b1.reshape(1, H), W2, b2.reshape(1, 1))


# speedup vs baseline: 1.0876x; 1.0290x over previous
import jax
import jax.numpy as jnp
from jax import lax
from jax.experimental import pallas as pl
from jax.experimental.pallas import tpu as pltpu

_G = 64
_CH = 2048


def _fused_body(x_ref, b_ref, W1_ref, b1_ref, W2_ref, b2_ref,
                out_ref, E_ref, S_ref, st_ref, en_ref, ri_ref):
    i = pl.program_id(0)
    nb = pl.num_programs(0)
    BLK = x_ref.shape[0]

    @pl.when(i == 0)
    def _init():
        out_ref[...] = jnp.zeros_like(out_ref)
        E_ref[...] = jnp.zeros_like(E_ref)
        S_ref[...] = jnp.zeros_like(S_ref)
        giota = lax.broadcasted_iota(jnp.int32, (_G, 1), 0)
        nch = b_ref.shape[0]
        cnt = jnp.zeros((_G, 1), jnp.float32)
        for c in range(nch):
            bc = b_ref[pl.ds(c, 1), :]
            eq = (giota == bc).astype(jnp.float32)
            cnt = cnt + jnp.sum(eq, axis=1, keepdims=True)
        gk = lax.broadcasted_iota(jnp.int32, (_G, _G), 0)
        gg = lax.broadcasted_iota(jnp.int32, (_G, _G), 1)
        tri_lt = (gk < gg).astype(jnp.float32)
        tri_le = (gk <= gg).astype(jnp.float32)
        st_ref[...] = jnp.sum(tri_lt * cnt, axis=0, keepdims=True)
        en_ref[...] = jnp.sum(tri_le * cnt, axis=0, keepdims=True)
        ri_ref[...] = lax.broadcasted_iota(
            jnp.int32, (BLK, _G), 0).astype(jnp.float32)

    xb = x_ref[...]
    h = jnp.tanh(jnp.dot(xb, W1_ref[...],
                         preferred_element_type=jnp.float32) + b1_ref[...])
    s = jnp.dot(h, W2_ref[...],
                preferred_element_type=jnp.float32) + b2_ref[...]
    e = jnp.exp(s)
    base = (jnp.float32(1.0) * i * BLK) * jnp.ones((1, _G), jnp.float32)
    r = ri_ref[...]
    onehot = ((r >= st_ref[...] - base) & (r < en_ref[...] - base)
              ).astype(jnp.float32)
    oe = onehot * e
    out_ref[...] += lax.dot_general(
        oe, xb, (((0,), (0,)), ((), ())),
        preferred_element_type=jnp.float32)
    S_ref[...] += lax.dot_general(
        onehot, s, (((0,), (0,)), ((), ())),
        preferred_element_type=jnp.float32)
    E_ref[...] += lax.dot_general(
        onehot, e, (((0,), (0,)), ((), ())),
        preferred_element_type=jnp.float32)

    @pl.when(i == nb - 1)
    def _fin():
        em = jnp.exp(-S_ref[...])
        out_ref[...] = (em * out_ref[...]) / (em * E_ref[...] + 1e-8)


def kernel(x, batch, W1, b1, W2, b2):
    N, D = x.shape
    H = W1.shape[1]
    BLK = 5000
    assert N % BLK == 0
    nb = N // BLK
    npad = -N % _CH
    b2d = jnp.pad(batch.astype(jnp.int32), (0, npad),
                  constant_values=jnp.int32(2 ** 30)).reshape(-1, _CH)
    nch = b2d.shape[0]
    return pl.pallas_call(
        _fused_body,
        grid=(nb,),
        in_specs=[
            pl.BlockSpec((BLK, D), lambda i: (i, 0)),
            pl.BlockSpec((nch, _CH), lambda i: (0, 0)),
            pl.BlockSpec((D, H), lambda i: (0, 0)),
            pl.BlockSpec((1, H), lambda i: (0, 0)),
            pl.BlockSpec((H, 1), lambda i: (0, 0)),
            pl.BlockSpec((1, 1), lambda i: (0, 0)),
        ],
        out_specs=pl.BlockSpec((_G, D), lambda i: (0, 0)),
        out_shape=jax.ShapeDtypeStruct((_G, D), jnp.float32),
        scratch_shapes=[
            pltpu.VMEM((_G, 1), jnp.float32),
            pltpu.VMEM((_G, 1), jnp.float32),
            pltpu.VMEM((1, _G), jnp.float32),
            pltpu.VMEM((1, _G), jnp.float32),
            pltpu.VMEM((5000, _G), jnp.float32),
        ],
    )(x, b2d, W1, b1.reshape(1, H), W2, b2.reshape(1, 1))


# speedup vs baseline: 1.3622x; 1.2524x over previous
import jax
import jax.numpy as jnp
from jax import lax
from jax.experimental import pallas as pl
from jax.experimental.pallas import tpu as pltpu

_G = 64
_CH = 2048


def _fused_body(x_ref, b_ref, W1T_ref, b1_ref, W2r_ref, b2_ref,
                out_ref, E_ref, S_ref, st_ref, en_ref):
    i = pl.program_id(0)
    nb = pl.num_programs(0)
    BLK = x_ref.shape[0]

    @pl.when(i == 0)
    def _init():
        out_ref[...] = jnp.zeros_like(out_ref)
        E_ref[...] = jnp.zeros_like(E_ref)
        S_ref[...] = jnp.zeros_like(S_ref)
        giota = lax.broadcasted_iota(jnp.int32, (_G, 1), 0)
        nch = b_ref.shape[0]
        cnt = jnp.zeros((_G, 1), jnp.float32)
        for c in range(nch):
            bc = b_ref[pl.ds(c, 1), :]
            eq = (giota == bc).astype(jnp.float32)
            cnt = cnt + jnp.sum(eq, axis=1, keepdims=True)
        gk = lax.broadcasted_iota(jnp.int32, (_G, _G), 0)
        gg = lax.broadcasted_iota(jnp.int32, (_G, _G), 1)
        tri_lt = (gk < gg).astype(jnp.float32)
        tri_le = (gk <= gg).astype(jnp.float32)
        st_row = jnp.sum(tri_lt * cnt, axis=0, keepdims=True)
        en_row = jnp.sum(tri_le * cnt, axis=0, keepdims=True)
        eye = (gk == gg).astype(jnp.float32)
        st_ref[...] = jnp.sum(eye * st_row, axis=1, keepdims=True)
        en_ref[...] = jnp.sum(eye * en_row, axis=1, keepdims=True)

    xb = x_ref[...]
    hT = jnp.tanh(
        lax.dot_general(W1T_ref[...], xb, (((1,), (1,)), ((), ())),
                        preferred_element_type=jnp.float32)
        + b1_ref[...])
    sT = (jnp.dot(W2r_ref[...], hT, preferred_element_type=jnp.float32)
          + b2_ref[...])
    eT = jnp.exp(sT)
    fbase = jnp.float32(1.0) * i * BLK
    r = fbase + lax.broadcasted_iota(
        jnp.int32, (1, BLK), 1).astype(jnp.float32)
    onehotT = ((r >= st_ref[...]) & (r < en_ref[...])
               ).astype(jnp.float32)
    oeT = onehotT * eT
    out_ref[...] += lax.dot_general(
        oeT, xb, (((1,), (0,)), ((), ())),
        preferred_element_type=jnp.float32)
    S_ref[...] += lax.dot_general(
        onehotT, sT, (((1,), (1,)), ((), ())),
        preferred_element_type=jnp.float32)
    E_ref[...] += lax.dot_general(
        onehotT, eT, (((1,), (1,)), ((), ())),
        preferred_element_type=jnp.float32)

    @pl.when(i == nb - 1)
    def _fin():
        em = jnp.exp(-S_ref[...])
        out_ref[...] = (em * out_ref[...]) / (em * E_ref[...] + 1e-8)


def kernel(x, batch, W1, b1, W2, b2):
    N, D = x.shape
    H = W1.shape[1]
    BLK = 5000
    assert N % BLK == 0
    nb = N // BLK
    npad = -N % _CH
    b2d = jnp.pad(batch.astype(jnp.int32), (0, npad),
                  constant_values=jnp.int32(2 ** 30)).reshape(-1, _CH)
    nch = b2d.shape[0]
    return pl.pallas_call(
        _fused_body,
        grid=(nb,),
        in_specs=[
            pl.BlockSpec((BLK, D), lambda i: (i, 0)),
            pl.BlockSpec((nch, _CH), lambda i: (0, 0)),
            pl.BlockSpec((H, D), lambda i: (0, 0)),
            pl.BlockSpec((H, 1), lambda i: (0, 0)),
            pl.BlockSpec((1, H), lambda i: (0, 0)),
            pl.BlockSpec((1, 1), lambda i: (0, 0)),
        ],
        out_specs=pl.BlockSpec((_G, D), lambda i: (0, 0)),
        out_shape=jax.ShapeDtypeStruct((_G, D), jnp.float32),
        scratch_shapes=[
            pltpu.VMEM((_G, 1), jnp.float32),
            pltpu.VMEM((_G, 1), jnp.float32),
            pltpu.VMEM((_G, 1), jnp.float32),
            pltpu.VMEM((_G, 1), jnp.float32),
        ],
    )(x, b2d, W1.T, b1.reshape(H, 1), W2.reshape(1, H), b2.reshape(1, 1))


# speedup vs baseline: 1.4213x; 1.0434x over previous
import jax
import jax.numpy as jnp
from jax import lax
from jax.experimental import pallas as pl
from jax.experimental.pallas import tpu as pltpu

_G = 64
_CH = 2048


def _fused_body(x_ref, b_ref, W1T_ref, b1_ref, W2r_ref, b2_ref,
                out_ref, E_ref, S_ref, st_ref, en_ref):
    i = pl.program_id(0)
    nb = pl.num_programs(0)
    BLK = x_ref.shape[0]

    @pl.when(i == 0)
    def _init():
        out_ref[...] = jnp.zeros_like(out_ref)
        E_ref[...] = jnp.zeros_like(E_ref)
        S_ref[...] = jnp.zeros_like(S_ref)
        giota = lax.broadcasted_iota(jnp.int32, (_G, 1), 0)
        NN = b_ref.shape[0]
        cnt = jnp.zeros((_G, 1), jnp.float32)
        for off in range(0, NN, _CH):
            sz = min(_CH, NN - off)
            bc = b_ref[pl.ds(off, sz)].reshape(1, sz)
            eq = (giota == bc).astype(jnp.float32)
            cnt = cnt + jnp.sum(eq, axis=1, keepdims=True)
        gk = lax.broadcasted_iota(jnp.int32, (_G, _G), 0)
        gg = lax.broadcasted_iota(jnp.int32, (_G, _G), 1)
        tri_lt = (gk < gg).astype(jnp.float32)
        tri_le = (gk <= gg).astype(jnp.float32)
        st_row = jnp.sum(tri_lt * cnt, axis=0, keepdims=True)
        en_row = jnp.sum(tri_le * cnt, axis=0, keepdims=True)
        eye = (gk == gg).astype(jnp.float32)
        st_ref[...] = jnp.sum(eye * st_row, axis=1, keepdims=True)
        en_ref[...] = jnp.sum(eye * en_row, axis=1, keepdims=True)

    xb = x_ref[...]
    hT = jnp.tanh(
        lax.dot_general(W1T_ref[...], xb, (((1,), (1,)), ((), ())),
                        preferred_element_type=jnp.float32)
        + b1_ref[...])
    sT = (jnp.dot(W2r_ref[...], hT, preferred_element_type=jnp.float32)
          + b2_ref[...])
    eT = jnp.exp(sT)
    fbase = jnp.float32(1.0) * i * BLK
    r = fbase + lax.broadcasted_iota(
        jnp.int32, (1, BLK), 1).astype(jnp.float32)
    onehotT = ((r >= st_ref[...]) & (r < en_ref[...])
               ).astype(jnp.float32)
    oeT = onehotT * eT
    out_ref[...] += lax.dot_general(
        oeT, xb, (((1,), (0,)), ((), ())),
        preferred_element_type=jnp.float32)
    S_ref[...] += lax.dot_general(
        onehotT, sT, (((1,), (1,)), ((), ())),
        preferred_element_type=jnp.float32)
    E_ref[...] += lax.dot_general(
        onehotT, eT, (((1,), (1,)), ((), ())),
        preferred_element_type=jnp.float32)

    @pl.when(i == nb - 1)
    def _fin():
        em = jnp.exp(-S_ref[...])
        out_ref[...] = (em * out_ref[...]) / (em * E_ref[...] + 1e-8)


def kernel(x, batch, W1, b1, W2, b2):
    N, D = x.shape
    H = W1.shape[1]
    BLK = 5000
    assert N % BLK == 0
    nb = N // BLK

    return pl.pallas_call(
        _fused_body,
        grid=(nb,),
        in_specs=[
            pl.BlockSpec((BLK, D), lambda i: (i, 0)),
            pl.BlockSpec((N,), lambda i: (0,)),
            pl.BlockSpec((H, D), lambda i: (0, 0)),
            pl.BlockSpec((H, 1), lambda i: (0, 0)),
            pl.BlockSpec((1, H), lambda i: (0, 0)),
            pl.BlockSpec((1, 1), lambda i: (0, 0)),
        ],
        out_specs=pl.BlockSpec((_G, D), lambda i: (0, 0)),
        out_shape=jax.ShapeDtypeStruct((_G, D), jnp.float32),
        scratch_shapes=[
            pltpu.VMEM((_G, 1), jnp.float32),
            pltpu.VMEM((_G, 1), jnp.float32),
            pltpu.VMEM((_G, 1), jnp.float32),
            pltpu.VMEM((_G, 1), jnp.float32),
        ],
    )(x, batch.astype(jnp.int32), W1.T, b1.reshape(H, 1), W2.reshape(1, H),
      b2.reshape(1, 1))


# speedup vs baseline: 1.4769x; 1.0391x over previous
import jax
import jax.numpy as jnp
from jax import lax
from jax.experimental import pallas as pl
from jax.experimental.pallas import tpu as pltpu

_G = 64
_CH = 2048


def _fused_body(x_ref, b_ref, W1_ref, b1_ref, W2_ref, b2_ref,
                out_ref, E_ref, S_ref, st_ref, en_ref,
                w1t_ref, b1c_ref, w2r_ref):
    i = pl.program_id(0)
    nb = pl.num_programs(0)
    BLK = x_ref.shape[0]

    @pl.when(i == 0)
    def _init():
        out_ref[...] = jnp.zeros_like(out_ref)
        E_ref[...] = jnp.zeros_like(E_ref)
        S_ref[...] = jnp.zeros_like(S_ref)
        giota = lax.broadcasted_iota(jnp.int32, (_G, 1), 0)
        NN = b_ref.shape[0]
        cnt = jnp.zeros((_G, 1), jnp.float32)
        for off in range(0, NN, _CH):
            sz = min(_CH, NN - off)
            bc = b_ref[pl.ds(off, sz)].reshape(1, sz)
            eq = (giota == bc).astype(jnp.float32)
            cnt = cnt + jnp.sum(eq, axis=1, keepdims=True)
        gk = lax.broadcasted_iota(jnp.int32, (_G, _G), 0)
        gg = lax.broadcasted_iota(jnp.int32, (_G, _G), 1)
        tri_lt = (gk < gg).astype(jnp.float32)
        tri_le = (gk <= gg).astype(jnp.float32)
        st_row = jnp.sum(tri_lt * cnt, axis=0, keepdims=True)
        en_row = jnp.sum(tri_le * cnt, axis=0, keepdims=True)
        eye = (gk == gg).astype(jnp.float32)
        st_ref[...] = jnp.sum(eye * st_row, axis=1, keepdims=True)
        en_ref[...] = jnp.sum(eye * en_row, axis=1, keepdims=True)
        w1t_ref[...] = W1_ref[...].T
        H = W1_ref.shape[1]
        hk = lax.broadcasted_iota(jnp.int32, (H, H), 0)
        hh = lax.broadcasted_iota(jnp.int32, (H, H), 1)
        heye = (hk == hh).astype(jnp.float32)
        b1c_ref[...] = jnp.sum(heye * b1_ref[...], axis=1, keepdims=True)
        w2r_ref[...] = jnp.sum(heye * W2_ref[...], axis=0, keepdims=True)

    xb = x_ref[...]
    hT = jnp.tanh(
        lax.dot_general(w1t_ref[...], xb, (((1,), (1,)), ((), ())),
                        preferred_element_type=jnp.float32)
        + b1c_ref[...])
    sT = (jnp.dot(w2r_ref[...], hT, preferred_element_type=jnp.float32)
          + b2_ref[...])
    eT = jnp.exp(sT)
    fbase = jnp.float32(1.0) * i * BLK
    r = fbase + lax.broadcasted_iota(
        jnp.int32, (1, BLK), 1).astype(jnp.float32)
    onehotT = ((r >= st_ref[...]) & (r < en_ref[...])
               ).astype(jnp.float32)
    oeT = onehotT * eT
    out_ref[...] += lax.dot_general(
        oeT, xb, (((1,), (0,)), ((), ())),
        preferred_element_type=jnp.float32)
    S_ref[...] += lax.dot_general(
        onehotT, sT, (((1,), (1,)), ((), ())),
        preferred_element_type=jnp.float32)
    E_ref[...] += lax.dot_general(
        onehotT, eT, (((1,), (1,)), ((), ())),
        preferred_element_type=jnp.float32)

    @pl.when(i == nb - 1)
    def _fin():
        em = jnp.exp(-S_ref[...])
        out_ref[...] = (em * out_ref[...]) / (em * E_ref[...] + 1e-8)


def kernel(x, batch, W1, b1, W2, b2):
    N, D = x.shape
    H = W1.shape[1]
    BLK = 5000
    assert N % BLK == 0
    nb = N // BLK

    return pl.pallas_call(
        _fused_body,
        grid=(nb,),
        in_specs=[
            pl.BlockSpec((BLK, D), lambda i: (i, 0)),
            pl.BlockSpec((N,), lambda i: (0,)),
            pl.BlockSpec((D, H), lambda i: (0, 0)),
            pl.BlockSpec((1, H), lambda i: (0, 0)),
            pl.BlockSpec((H, 1), lambda i: (0, 0)),
            pl.BlockSpec((1, 1), lambda i: (0, 0)),
        ],
        out_specs=pl.BlockSpec((_G, D), lambda i: (0, 0)),
        out_shape=jax.ShapeDtypeStruct((_G, D), jnp.float32),
        scratch_shapes=[
            pltpu.VMEM((_G, 1), jnp.float32),
            pltpu.VMEM((_G, 1), jnp.float32),
            pltpu.VMEM((_G, 1), jnp.float32),
            pltpu.VMEM((_G, 1), jnp.float32),
            pltpu.VMEM((256, 512), jnp.float32),
            pltpu.VMEM((256, 1), jnp.float32),
            pltpu.VMEM((1, 256), jnp.float32),
        ],
    )(x, batch.astype(jnp.int32), W1, b1.reshape(1, H), W2, b2.reshape(1, 1))


# speedup vs baseline: 1.4826x; 1.0039x over previous
import jax
import jax.numpy as jnp
from jax import lax
from jax.experimental import pallas as pl
from jax.experimental.pallas import tpu as pltpu

_G = 64
_CH = 2048


def _fused_body(x_ref, b_ref, W1_ref, b1_ref, W2_ref, b2_ref,
                out_ref, E_ref, S_ref, st_ref, en_ref,
                w1t_ref, b1c_ref, w2r_ref):
    i = pl.program_id(0)
    nb = pl.num_programs(0)
    BLK = x_ref.shape[0]

    @pl.when(i == 0)
    def _init():
        out_ref[...] = jnp.zeros_like(out_ref)
        E_ref[...] = jnp.zeros_like(E_ref)
        S_ref[...] = jnp.zeros_like(S_ref)
        giota = lax.broadcasted_iota(jnp.int32, (_G, 1), 0)
        NN = b_ref.shape[0]
        cnt = jnp.zeros((_G, 1), jnp.float32)
        for off in range(0, NN, _CH):
            sz = min(_CH, NN - off)
            bc = b_ref[pl.ds(off, sz)].reshape(1, sz)
            eq = (giota == bc).astype(jnp.float32)
            cnt = cnt + jnp.sum(eq, axis=1, keepdims=True)
        gk = lax.broadcasted_iota(jnp.int32, (_G, _G), 0)
        gg = lax.broadcasted_iota(jnp.int32, (_G, _G), 1)
        tri_lt = (gk < gg).astype(jnp.float32)
        tri_le = (gk <= gg).astype(jnp.float32)
        st_row = jnp.sum(tri_lt * cnt, axis=0, keepdims=True)
        en_row = jnp.sum(tri_le * cnt, axis=0, keepdims=True)
        eye = (gk == gg).astype(jnp.float32)
        st_ref[...] = jnp.sum(eye * st_row, axis=1, keepdims=True)
        en_ref[...] = jnp.sum(eye * en_row, axis=1, keepdims=True)
        w1t_ref[...] = W1_ref[...].T
        H = W1_ref.shape[1]
        hk = lax.broadcasted_iota(jnp.int32, (H, H), 0)
        hh = lax.broadcasted_iota(jnp.int32, (H, H), 1)
        heye = (hk == hh).astype(jnp.float32)
        b1c_ref[...] = jnp.sum(heye * b1_ref[...], axis=1, keepdims=True)
        w2r_ref[...] = jnp.sum(heye * W2_ref[...], axis=0, keepdims=True)

    xb = x_ref[...]
    hT = jnp.tanh(
        lax.dot_general(w1t_ref[...], xb, (((1,), (1,)), ((), ())),
                        preferred_element_type=jnp.float32)
        + b1c_ref[...])
    sT = (jnp.dot(w2r_ref[...], hT, preferred_element_type=jnp.float32)
          + b2_ref[...])
    eT = jnp.exp(sT)
    fbase = jnp.float32(1.0) * i * BLK
    r = fbase + lax.broadcasted_iota(
        jnp.int32, (1, BLK), 1).astype(jnp.float32)
    onehotT = ((r >= st_ref[...]) & (r < en_ref[...])
               ).astype(jnp.float32)
    oeT = onehotT * eT
    out_ref[...] += lax.dot_general(
        oeT, xb, (((1,), (0,)), ((), ())),
        preferred_element_type=jnp.float32)
    S_ref[...] += lax.dot_general(
        onehotT, sT, (((1,), (1,)), ((), ())),
        preferred_element_type=jnp.float32)
    E_ref[...] += lax.dot_general(
        onehotT, eT, (((1,), (1,)), ((), ())),
        preferred_element_type=jnp.float32)

    @pl.when(i == nb - 1)
    def _fin():
        em = jnp.exp(-S_ref[...])
        out_ref[...] = (em * out_ref[...]) / (em * E_ref[...] + 1e-8)


def kernel(x, batch, W1, b1, W2, b2):
    N, D = x.shape
    H = W1.shape[1]
    BLK = 5000
    assert N % BLK == 0
    nb = N // BLK

    return pl.pallas_call(
        _fused_body,
        grid=(nb,),
        in_specs=[
            pl.BlockSpec((BLK, D), lambda i: (i, 0)),
            pl.BlockSpec((N,), lambda i: (0,)),
            pl.BlockSpec((D, H), lambda i: (0, 0)),
            pl.BlockSpec((1, H), lambda i: (0, 0)),
            pl.BlockSpec((H, 1), lambda i: (0, 0)),
            pl.BlockSpec((1, 1), lambda i: (0, 0)),
        ],
        out_specs=pl.BlockSpec((_G, D), lambda i: (0, 0)),
        out_shape=jax.ShapeDtypeStruct((_G, D), jnp.float32),
        scratch_shapes=[
            pltpu.VMEM((_G, 1), jnp.float32),
            pltpu.VMEM((_G, 1), jnp.float32),
            pltpu.VMEM((_G, 1), jnp.float32),
            pltpu.VMEM((_G, 1), jnp.float32),
            pltpu.VMEM((H, D), jnp.float32),
            pltpu.VMEM((H, 1), jnp.float32),
            pltpu.VMEM((1, H), jnp.float32),
        ],
    )(x, batch.astype(jnp.int32), W1, b1.reshape(1, H), W2, b2.reshape(1, 1))
